# baseline (speedup 1.0000x reference)
import functools

import jax
import jax.numpy as jnp
from jax import lax
from jax.experimental import pallas as pl
from jax.experimental.pallas import tpu as pltpu
from jax.experimental.pallas import tpu_sc as plsc

N = 50000
E = 800000
IN = 128
S = 64
Z = 32
G = 16
R = 6

NC = 2
NS = 16
HALF = 25000
HALF_PAD = 25088
TROWS = HALF_PAD // NS
TRASH = HALF_PAD - 1
CH = 32
E_PAD = 819200
EPT = E_PAD // NS
CPB = 64
RING = 12
LOOK = 10
IB = CH * CPB
NBI = EPT // IB
DROWS = E_PAD // CH
ZROWS = 16

_sc_mesh = plsc.VectorSubcoreMesh(core_axis_name="c", subcore_axis_name="s")

_TOTC = NC * NS * EPT


@functools.partial(
    pl.kernel,
    out_type=[
        jax.ShapeDtypeStruct((_TOTC,), jnp.int32),
        jax.ShapeDtypeStruct((_TOTC,), jnp.int32),
        jax.ShapeDtypeStruct((NC * NS * 16,), jnp.int32),
    ],
    mesh=_sc_mesh,
    scratch_types=[
        pltpu.VMEM((IB,), jnp.int32),
        pltpu.VMEM((IB,), jnp.int32),
        pltpu.VMEM((EPT + 16,), jnp.int32),
        pltpu.VMEM((EPT + 16,), jnp.int32),
    ],
    compiler_params=pltpu.CompilerParams(
        use_tc_tiling_on_sc=False, needs_layout_passes=False),
)
def _sc_compact(src_hbm, dl_hbm, csrc_hbm, cdst_hbm, cnt_hbm,
                srcst, dlst, csb, cdb):
    c = lax.axis_index("c")
    s = lax.axis_index("s")
    w = c * NS + s

    def _blk(b, off):
        pltpu.sync_copy(src_hbm.at[pl.ds(s * EPT + b * IB, IB)], srcst)
        pltpu.sync_copy(dl_hbm.at[pl.ds(c * E_PAD + s * EPT + b * IB, IB)],
                        dlst)

        def _grp(g, off2):
            s16 = srcst[pl.ds(g * 16, 16)]
            d16 = dlst[pl.ds(g * 16, 16)]
            keep = d16 != TRASH
            ki = jnp.where(keep, 1, 0)
            pos = jnp.where(keep, off2 + jnp.cumsum(ki) - 1, EPT)
            plsc.store_scatter(csb, [pos], s16)
            plsc.store_scatter(cdb, [pos], d16)
            return off2 + jnp.sum(ki)

        return lax.fori_loop(0, IB // 16, _grp, off)

    cnt = lax.fori_loop(0, NBI, _blk, jnp.int32(0))

    target = jnp.maximum((cnt + IB - 1) // IB, 1) * IB
    zpad = jnp.zeros((16,), jnp.int32)
    tpad = jnp.full((16,), TRASH, jnp.int32)

    def _pad(p, off2):
        csb[pl.ds(off2, 16)] = zpad
        cdb[pl.ds(off2, 16)] = tpad
        return off2 + 16

    lax.fori_loop(0, (target - cnt + 15) // 16, _pad, cnt)

    nblk = target // IB
    srcst[pl.ds(0, 16)] = jnp.full((16,), nblk, jnp.int32)
    pltpu.sync_copy(srcst.at[pl.ds(0, 16)], cnt_hbm.at[pl.ds(w * 16, 16)])
    pltpu.sync_copy(csb.at[pl.ds(0, EPT)], csrc_hbm.at[pl.ds(w * EPT, EPT)])
    pltpu.sync_copy(cdb.at[pl.ds(0, EPT)], cdst_hbm.at[pl.ds(w * EPT, EPT)])


@functools.partial(
    pl.kernel,
    out_type=jax.ShapeDtypeStruct((NC * HALF_PAD, S), jnp.float32),
    mesh=_sc_mesh,
    scratch_types=[
        pltpu.VMEM((IB,), jnp.int32),
        pltpu.VMEM((CPB, CH), jnp.int32),
        pltpu.VMEM((RING, CH, S), jnp.float32),
        pltpu.VMEM((16,), jnp.int32),
        pltpu.VMEM_SHARED((HALF_PAD, S), jnp.float32),
        pltpu.SemaphoreType.DMA((RING,)),
        pltpu.SemaphoreType.DMA((RING,)),
        pltpu.SemaphoreType.DMA,
    ],
    compiler_params=pltpu.CompilerParams(
        use_tc_tiling_on_sc=False, needs_layout_passes=False),
)
def _sc_agg(m_hbm, csrc_hbm, cdst_hbm, cnt_hbm, z_hbm, out_hbm, srcb, dstb,
            rows, cntv, aggsh, semg, semsc, semi):
    c = lax.axis_index("c")
    s = lax.axis_index("s")
    w = c * NS + s

    pltpu.sync_copy(z_hbm, aggsh.at[pl.ds(s * TROWS, TROWS)])
    pltpu.sync_copy(cnt_hbm.at[pl.ds(w * 16, 16)], cntv)
    plsc.subcore_barrier()
    nblk = jnp.max(cntv[...])

    ebase = w * EPT
    dlbase = w * (EPT // CH)

    def _fire_src(blk):
        pltpu.async_copy(csrc_hbm.at[pl.ds(ebase + blk * IB, IB)], srcb,
                         semi)

    def _fire_dst(blk):
        pltpu.async_copy(cdst_hbm.at[pl.ds(dlbase + blk * CPB, CPB)], dstb,
                         semi)

    def _wait_idx():
        pltpu.make_async_copy(csrc_hbm.at[pl.ds(0, IB)], srcb, semi).wait()
        pltpu.make_async_copy(cdst_hbm.at[pl.ds(0, CPB)], dstb, semi).wait()

    def _gather(k):
        q = k % RING
        return pltpu.async_copy(
            m_hbm.at[srcb.at[pl.ds(k * CH, CH)]], rows.at[q], semg.at[q])

    _fire_src(0)
    _fire_dst(0)

    def _blk_body(blk, carry):
        _wait_idx()
        descs = [None] * CPB
        g = [None] * CPB
        waited = set()
        for k in range(LOOK):
            g[k] = _gather(k)
        for k in range(CPB):
            q = k % RING
            g[k].wait()
            descs[k] = pltpu.async_copy(rows.at[q], aggsh.at[dstb.at[k]],
                                        semsc.at[q], add=True)
            if k + LOOK < CPB:
                if k + LOOK - RING >= 0:
                    descs[k + LOOK - RING].wait()
                    waited.add(k + LOOK - RING)
                g[k + LOOK] = _gather(k + LOOK)

        @pl.when(blk + 1 < nblk)
        def _():
            _fire_src(blk + 1)

        for k in range(CPB):
            if k not in waited:
                descs[k].wait()

        @pl.when(blk + 1 < nblk)
        def _():
            _fire_dst(blk + 1)

        return carry

    lax.fori_loop(0, nblk, _blk_body, 0)

    plsc.subcore_barrier()

    pltpu.sync_copy(
        aggsh.at[pl.ds(s * TROWS, TROWS)],
        out_hbm.at[pl.ds(c * HALF_PAD + s * TROWS, TROWS)],
    )


BN = 2000
NBLK = N // BN


def _tc_dl_body(d_ref, dl0_ref, dl1_ref):
    d = d_ref[...]
    dl0_ref[...] = jnp.where((d >= 0) & (d < HALF), d, TRASH)
    dl1_ref[...] = jnp.where(d >= HALF, d - HALF, TRASH)


def _tc_init_body(x_ref, wi_ref, bi_ref, wm_ref, bm_ref, h_ref, m_ref):
    h = jnp.maximum(
        jnp.dot(x_ref[...], wi_ref[...],
                preferred_element_type=jnp.float32) + bi_ref[...], 0.0)
    h_ref[...] = h
    m_ref[...] = jnp.maximum(
        jnp.dot(h, wm_ref[...], preferred_element_type=jnp.float32)
        + bm_ref[...], 0.0)


def _gru_from_agg(agg, h, wu, bu, wr, br, wz, bz, wn, bn, hr, hz, hn_w,
                  bhr, bhz, bhn):
    msg = jnp.maximum(
        jnp.dot(agg, wu, preferred_element_type=jnp.float32) + bu, 0.0)
    g_r = jax.nn.sigmoid(
        jnp.dot(msg, wr, preferred_element_type=jnp.float32) + br
        + jnp.dot(h, hr, preferred_element_type=jnp.float32) + bhr)
    g_z = jax.nn.sigmoid(
        jnp.dot(msg, wz, preferred_element_type=jnp.float32) + bz
        + jnp.dot(h, hz, preferred_element_type=jnp.float32) + bhz)
    g_n = jnp.tanh(
        jnp.dot(msg, wn, preferred_element_type=jnp.float32) + bn
        + g_r * (jnp.dot(h, hn_w, preferred_element_type=jnp.float32) + bhn))
    return (1.0 - g_z) * g_n + g_z * h


def _tc_round_body(agg_ref, h_ref, wu_ref, bu_ref, wr_ref, br_ref, wz_ref,
                   bz_ref, wn_ref, bn_ref, hr_ref, hz_ref, hn_ref, bhr_ref,
                   bhz_ref, bhn_ref, wm_ref, bm_ref, hout_ref, mout_ref):
    hn = _gru_from_agg(agg_ref[...], h_ref[...], wu_ref[...], bu_ref[...],
                       wr_ref[...], br_ref[...], wz_ref[...], bz_ref[...],
                       wn_ref[...], bn_ref[...], hr_ref[...], hz_ref[...],
                       hn_ref[...], bhr_ref[...], bhz_ref[...], bhn_ref[...])
    hout_ref[...] = hn
    mout_ref[...] = jnp.maximum(
        jnp.dot(hn, wm_ref[...], preferred_element_type=jnp.float32)
        + bm_ref[...], 0.0)


def _tc_last_body(agg_ref, h_ref, wu_ref, bu_ref, wr_ref, br_ref, wz_ref,
                  bz_ref, wn_ref, bn_ref, hr_ref, hz_ref, hn_ref, bhr_ref,
                  bhz_ref, bhn_ref, muw_ref, mub_ref, lvw_ref, lvb_ref,
                  gm1_ref, gb1_ref, gm2_ref, gb2_ref, gl1_ref, gc1_ref,
                  gl2_ref, gc2_ref, mu_ref, lv_ref, mug_ref, lvg_ref,
                  sum_ref):
    i = pl.program_id(0)
    hn = _gru_from_agg(agg_ref[...], h_ref[...], wu_ref[...], bu_ref[...],
                       wr_ref[...], br_ref[...], wz_ref[...], bz_ref[...],
                       wn_ref[...], bn_ref[...], hr_ref[...], hz_ref[...],
                       hn_ref[...], bhr_ref[...], bhz_ref[...], bhn_ref[...])
    mu_ref[...] = jnp.dot(
        hn, muw_ref[...], preferred_element_type=jnp.float32) + mub_ref[...]
    lv_ref[...] = jnp.dot(
        hn, lvw_ref[...], preferred_element_type=jnp.float32) + lvb_ref[...]

    bsum = jnp.sum(hn, axis=0, keepdims=True)

    @pl.when(i == 0)
    def _():
        sum_ref[...] = jnp.zeros_like(sum_ref)

    sum_ref[...] += jnp.broadcast_to(bsum, sum_ref.shape)

    @pl.when(i == NBLK - 1)
    def _():
        g = sum_ref[0:1, :] * (1.0 / N)
        gmu = jnp.dot(
            jnp.maximum(
                jnp.dot(g, gm1_ref[...],
                        preferred_element_type=jnp.float32) + gb1_ref[...],
                0.0),
            gm2_ref[...], preferred_element_type=jnp.float32) + gb2_ref[...]
        glv = jnp.dot(
            jnp.maximum(
                jnp.dot(g, gl1_ref[...],
                        preferred_element_type=jnp.float32) + gc1_ref[...],
                0.0),
            gl2_ref[...], preferred_element_type=jnp.float32) + gc2_ref[...]
        mug_ref[...] = jnp.broadcast_to(gmu, mug_ref.shape)
        lvg_ref[...] = jnp.broadcast_to(glv, lvg_ref.shape)


def _row_spec(cols):
    return pl.BlockSpec((BN, cols), lambda i: (i, 0))


def _w_spec(shape):
    nd = len(shape)
    return pl.BlockSpec(shape, lambda i, _nd=nd: (0,) * _nd)


def kernel(x, edge_index, input_W, input_b, msg_W, msg_b, upd_W, upd_b,
           gru_Wih, gru_Whh, gru_bih, gru_bhh, mu_W, mu_b, lv_W, lv_b,
           gmu_W1, gmu_b1, gmu_W2, gmu_b2, glv_W1, glv_b1, glv_W2, glv_b2):
    f32 = jnp.float32
    src = edge_index[0]
    dst = edge_index[1]

    wih_t = gru_Wih.T
    whh_t = gru_Whh.T
    wr, wz, wn = wih_t[:, :S], wih_t[:, S:2 * S], wih_t[:, 2 * S:]
    hr, hz, hn = whh_t[:, :S], whh_t[:, S:2 * S], whh_t[:, 2 * S:]
    br = gru_bih[:S].reshape(1, S)
    bz = gru_bih[S:2 * S].reshape(1, S)
    bn = gru_bih[2 * S:].reshape(1, S)
    bhr = gru_bhh[:S].reshape(1, S)
    bhz = gru_bhh[S:2 * S].reshape(1, S)
    bhn = gru_bhh[2 * S:].reshape(1, S)

    src_pad = jnp.concatenate(
        [src, jnp.zeros((E_PAD - E,), jnp.int32)])
    dst_pad = jnp.concatenate(
        [dst, jnp.full((E_PAD - E,), -1, jnp.int32)])

    dst2 = dst_pad.reshape(640, 1280)
    dl0, dl1 = pl.pallas_call(
        _tc_dl_body,
        grid=(1,),
        in_specs=[pl.BlockSpec((640, 1280), lambda i: (0, 0))],
        out_specs=[pl.BlockSpec((640, 1280), lambda i: (0, 0))] * 2,
        out_shape=[jax.ShapeDtypeStruct((640, 1280), jnp.int32)] * 2,
    )(dst2)
    dl_flat = jnp.concatenate([dl0.reshape(-1), dl1.reshape(-1)])

    csrc, cdst_flat, cnts = _sc_compact(src_pad, dl_flat)
    cdst2 = cdst_flat.reshape(NC * NS * (EPT // CH), CH)
    zrows = jnp.zeros((TROWS, S), jnp.float32)

    h, m = pl.pallas_call(
        _tc_init_body,
        grid=(NBLK,),
        in_specs=[
            _row_spec(IN),
            _w_spec((IN, S)), _w_spec((1, S)),
            _w_spec((S, S)), _w_spec((1, S)),
        ],
        out_specs=[_row_spec(S), _row_spec(S)],
        out_shape=[
            jax.ShapeDtypeStruct((N, S), f32),
            jax.ShapeDtypeStruct((N, S), f32),
        ],
    )(x, input_W, input_b.reshape(1, S), msg_W[0], msg_b[0].reshape(1, S))

    round_specs = (
        [_row_spec(S), _row_spec(S)]
        + [_w_spec((S, S)), _w_spec((1, S))] * 4
        + [_w_spec((S, S))] * 3
        + [_w_spec((1, S))] * 3
    )

    for r in range(R):
        agg_pad = _sc_agg(m, csrc, cdst2, cnts, zrows)
        agg = jnp.concatenate(
            [agg_pad[:HALF], agg_pad[HALF_PAD:HALF_PAD + HALF]], axis=0)
        round_args = (
            agg, h,
            upd_W[r], upd_b[r].reshape(1, S),
            wr, br, wz, bz, wn, bn,
            hr, hz, hn, bhr, bhz, bhn,
        )
        if r < R - 1:
            h, m = pl.pallas_call(
                _tc_round_body,
                grid=(NBLK,),
                in_specs=round_specs + [_w_spec((S, S)), _w_spec((1, S))],
                out_specs=[_row_spec(S), _row_spec(S)],
                out_shape=[
                    jax.ShapeDtypeStruct((N, S), f32),
                    jax.ShapeDtypeStruct((N, S), f32),
                ],
            )(*round_args, msg_W[r + 1], msg_b[r + 1].reshape(1, S))
        else:
            mu_node, lv_node, mu_g8, lv_g8 = pl.pallas_call(
                _tc_last_body,
                grid=(NBLK,),
                in_specs=round_specs + [
                    _w_spec((S, Z)), _w_spec((1, Z)),
                    _w_spec((S, Z)), _w_spec((1, Z)),
                    _w_spec((S, S)), _w_spec((1, S)),
                    _w_spec((S, G)), _w_spec((1, G)),
                    _w_spec((S, S)), _w_spec((1, S)),
                    _w_spec((S, G)), _w_spec((1, G)),
                ],
                out_specs=[
                    _row_spec(Z), _row_spec(Z),
                    pl.BlockSpec((8, G), lambda i: (0, 0)),
                    pl.BlockSpec((8, G), lambda i: (0, 0)),
                ],
                out_shape=[
                    jax.ShapeDtypeStruct((N, Z), f32),
                    jax.ShapeDtypeStruct((N, Z), f32),
                    jax.ShapeDtypeStruct((8, G), f32),
                    jax.ShapeDtypeStruct((8, G), f32),
                ],
                scratch_shapes=[pltpu.VMEM((8, S), f32)],
            )(*round_args,
              mu_W, mu_b.reshape(1, Z), lv_W, lv_b.reshape(1, Z),
              gmu_W1, gmu_b1.reshape(1, S), gmu_W2, gmu_b2.reshape(1, G),
              glv_W1, glv_b1.reshape(1, S), glv_W2, glv_b2.reshape(1, G))

    return (mu_node, lv_node, mu_g8[0], lv_g8[0])


# speedup vs baseline: 1.0040x; 1.0040x over previous
import functools

import jax
import jax.numpy as jnp
from jax import lax
from jax.experimental import pallas as pl
from jax.experimental.pallas import tpu as pltpu
from jax.experimental.pallas import tpu_sc as plsc

N = 50000
E = 800000
IN = 128
S = 64
Z = 32
G = 16
R = 6

NC = 2
NS = 16
HALF = 25000
HALF_PAD = 25088
TROWS = HALF_PAD // NS
TRASH = HALF_PAD - 1
CH = 64
E_PAD = 819200
EPT = E_PAD // NS
CPB = 32
RING = 6
LOOK = 5
IB = CH * CPB
NBI = EPT // IB
DROWS = E_PAD // CH
ZROWS = 16

_sc_mesh = plsc.VectorSubcoreMesh(core_axis_name="c", subcore_axis_name="s")

_TOTC = NC * NS * EPT


@functools.partial(
    pl.kernel,
    out_type=[
        jax.ShapeDtypeStruct((_TOTC,), jnp.int32),
        jax.ShapeDtypeStruct((_TOTC,), jnp.int32),
        jax.ShapeDtypeStruct((NC * NS * 16,), jnp.int32),
    ],
    mesh=_sc_mesh,
    scratch_types=[
        pltpu.VMEM((IB,), jnp.int32),
        pltpu.VMEM((IB,), jnp.int32),
        pltpu.VMEM((EPT + 16,), jnp.int32),
        pltpu.VMEM((EPT + 16,), jnp.int32),
    ],
    compiler_params=pltpu.CompilerParams(
        use_tc_tiling_on_sc=False, needs_layout_passes=False),
)
def _sc_compact(src_hbm, dl_hbm, csrc_hbm, cdst_hbm, cnt_hbm,
                srcst, dlst, csb, cdb):
    c = lax.axis_index("c")
    s = lax.axis_index("s")
    w = c * NS + s

    def _blk(b, off):
        pltpu.sync_copy(src_hbm.at[pl.ds(s * EPT + b * IB, IB)], srcst)
        pltpu.sync_copy(dl_hbm.at[pl.ds(c * E_PAD + s * EPT + b * IB, IB)],
                        dlst)

        def _grp(g, off2):
            s16 = srcst[pl.ds(g * 16, 16)]
            d16 = dlst[pl.ds(g * 16, 16)]
            keep = d16 != TRASH
            ki = jnp.where(keep, 1, 0)
            pos = jnp.where(keep, off2 + jnp.cumsum(ki) - 1, EPT)
            plsc.store_scatter(csb, [pos], s16)
            plsc.store_scatter(cdb, [pos], d16)
            return off2 + jnp.sum(ki)

        return lax.fori_loop(0, IB // 16, _grp, off)

    cnt = lax.fori_loop(0, NBI, _blk, jnp.int32(0))

    target = jnp.maximum((cnt + IB - 1) // IB, 1) * IB
    zpad = jnp.zeros((16,), jnp.int32)
    tpad = jnp.full((16,), TRASH, jnp.int32)

    def _pad(p, off2):
        csb[pl.ds(off2, 16)] = zpad
        cdb[pl.ds(off2, 16)] = tpad
        return off2 + 16

    lax.fori_loop(0, (target - cnt + 15) // 16, _pad, cnt)

    nblk = target // IB
    srcst[pl.ds(0, 16)] = jnp.full((16,), nblk, jnp.int32)
    pltpu.sync_copy(srcst.at[pl.ds(0, 16)], cnt_hbm.at[pl.ds(w * 16, 16)])
    pltpu.sync_copy(csb.at[pl.ds(0, EPT)], csrc_hbm.at[pl.ds(w * EPT, EPT)])
    pltpu.sync_copy(cdb.at[pl.ds(0, EPT)], cdst_hbm.at[pl.ds(w * EPT, EPT)])


@functools.partial(
    pl.kernel,
    out_type=jax.ShapeDtypeStruct((NC * HALF_PAD, S), jnp.float32),
    mesh=_sc_mesh,
    scratch_types=[
        pltpu.VMEM((IB,), jnp.int32),
        pltpu.VMEM((CPB, CH), jnp.int32),
        pltpu.VMEM((RING, CH, S), jnp.float32),
        pltpu.VMEM((16,), jnp.int32),
        pltpu.VMEM_SHARED((HALF_PAD, S), jnp.float32),
        pltpu.SemaphoreType.DMA((RING,)),
        pltpu.SemaphoreType.DMA((RING,)),
        pltpu.SemaphoreType.DMA,
    ],
    compiler_params=pltpu.CompilerParams(
        use_tc_tiling_on_sc=False, needs_layout_passes=False),
)
def _sc_agg(m_hbm, csrc_hbm, cdst_hbm, cnt_hbm, z_hbm, out_hbm, srcb, dstb,
            rows, cntv, aggsh, semg, semsc, semi):
    c = lax.axis_index("c")
    s = lax.axis_index("s")
    w = c * NS + s

    pltpu.sync_copy(z_hbm, aggsh.at[pl.ds(s * TROWS, TROWS)])
    pltpu.sync_copy(cnt_hbm.at[pl.ds(w * 16, 16)], cntv)
    plsc.subcore_barrier()
    nblk = jnp.max(cntv[...])

    ebase = w * EPT
    dlbase = w * (EPT // CH)

    def _fire_src(blk):
        pltpu.async_copy(csrc_hbm.at[pl.ds(ebase + blk * IB, IB)], srcb,
                         semi)

    def _fire_dst(blk):
        pltpu.async_copy(cdst_hbm.at[pl.ds(dlbase + blk * CPB, CPB)], dstb,
                         semi)

    def _wait_idx():
        pltpu.make_async_copy(csrc_hbm.at[pl.ds(0, IB)], srcb, semi).wait()
        pltpu.make_async_copy(cdst_hbm.at[pl.ds(0, CPB)], dstb, semi).wait()

    def _gather(k):
        q = k % RING
        return pltpu.async_copy(
            m_hbm.at[srcb.at[pl.ds(k * CH, CH)]], rows.at[q], semg.at[q])

    _fire_src(0)
    _fire_dst(0)

    def _blk_body(blk, carry):
        _wait_idx()
        descs = [None] * CPB
        g = [None] * CPB
        waited = set()
        for k in range(LOOK):
            g[k] = _gather(k)
        for k in range(CPB):
            q = k % RING
            g[k].wait()
            descs[k] = pltpu.async_copy(rows.at[q], aggsh.at[dstb.at[k]],
                                        semsc.at[q], add=True)
            if k + LOOK < CPB:
                if k + LOOK - RING >= 0:
                    descs[k + LOOK - RING].wait()
                    waited.add(k + LOOK - RING)
                g[k + LOOK] = _gather(k + LOOK)

        @pl.when(blk + 1 < nblk)
        def _():
            _fire_src(blk + 1)

        for k in range(CPB):
            if k not in waited:
                descs[k].wait()

        @pl.when(blk + 1 < nblk)
        def _():
            _fire_dst(blk + 1)

        return carry

    lax.fori_loop(0, nblk, _blk_body, 0)

    plsc.subcore_barrier()

    pltpu.sync_copy(
        aggsh.at[pl.ds(s * TROWS, TROWS)],
        out_hbm.at[pl.ds(c * HALF_PAD + s * TROWS, TROWS)],
    )


BN = 2000
NBLK = N // BN


def _tc_dl_body(d_ref, dl0_ref, dl1_ref):
    d = d_ref[...]
    dl0_ref[...] = jnp.where((d >= 0) & (d < HALF), d, TRASH)
    dl1_ref[...] = jnp.where(d >= HALF, d - HALF, TRASH)


def _tc_init_body(x_ref, wi_ref, bi_ref, wm_ref, bm_ref, h_ref, m_ref):
    h = jnp.maximum(
        jnp.dot(x_ref[...], wi_ref[...],
                preferred_element_type=jnp.float32) + bi_ref[...], 0.0)
    h_ref[...] = h
    m_ref[...] = jnp.maximum(
        jnp.dot(h, wm_ref[...], preferred_element_type=jnp.float32)
        + bm_ref[...], 0.0)


def _gru_from_agg(agg, h, wu, bu, wr, br, wz, bz, wn, bn, hr, hz, hn_w,
                  bhr, bhz, bhn):
    msg = jnp.maximum(
        jnp.dot(agg, wu, preferred_element_type=jnp.float32) + bu, 0.0)
    g_r = jax.nn.sigmoid(
        jnp.dot(msg, wr, preferred_element_type=jnp.float32) + br
        + jnp.dot(h, hr, preferred_element_type=jnp.float32) + bhr)
    g_z = jax.nn.sigmoid(
        jnp.dot(msg, wz, preferred_element_type=jnp.float32) + bz
        + jnp.dot(h, hz, preferred_element_type=jnp.float32) + bhz)
    g_n = jnp.tanh(
        jnp.dot(msg, wn, preferred_element_type=jnp.float32) + bn
        + g_r * (jnp.dot(h, hn_w, preferred_element_type=jnp.float32) + bhn))
    return (1.0 - g_z) * g_n + g_z * h


def _tc_round_body(agg_ref, h_ref, wu_ref, bu_ref, wr_ref, br_ref, wz_ref,
                   bz_ref, wn_ref, bn_ref, hr_ref, hz_ref, hn_ref, bhr_ref,
                   bhz_ref, bhn_ref, wm_ref, bm_ref, hout_ref, mout_ref):
    hn = _gru_from_agg(agg_ref[...], h_ref[...], wu_ref[...], bu_ref[...],
                       wr_ref[...], br_ref[...], wz_ref[...], bz_ref[...],
                       wn_ref[...], bn_ref[...], hr_ref[...], hz_ref[...],
                       hn_ref[...], bhr_ref[...], bhz_ref[...], bhn_ref[...])
    hout_ref[...] = hn
    mout_ref[...] = jnp.maximum(
        jnp.dot(hn, wm_ref[...], preferred_element_type=jnp.float32)
        + bm_ref[...], 0.0)


def _tc_last_body(agg_ref, h_ref, wu_ref, bu_ref, wr_ref, br_ref, wz_ref,
                  bz_ref, wn_ref, bn_ref, hr_ref, hz_ref, hn_ref, bhr_ref,
                  bhz_ref, bhn_ref, muw_ref, mub_ref, lvw_ref, lvb_ref,
                  gm1_ref, gb1_ref, gm2_ref, gb2_ref, gl1_ref, gc1_ref,
                  gl2_ref, gc2_ref, mu_ref, lv_ref, mug_ref, lvg_ref,
                  sum_ref):
    i = pl.program_id(0)
    hn = _gru_from_agg(agg_ref[...], h_ref[...], wu_ref[...], bu_ref[...],
                       wr_ref[...], br_ref[...], wz_ref[...], bz_ref[...],
                       wn_ref[...], bn_ref[...], hr_ref[...], hz_ref[...],
                       hn_ref[...], bhr_ref[...], bhz_ref[...], bhn_ref[...])
    mu_ref[...] = jnp.dot(
        hn, muw_ref[...], preferred_element_type=jnp.float32) + mub_ref[...]
    lv_ref[...] = jnp.dot(
        hn, lvw_ref[...], preferred_element_type=jnp.float32) + lvb_ref[...]

    bsum = jnp.sum(hn, axis=0, keepdims=True)

    @pl.when(i == 0)
    def _():
        sum_ref[...] = jnp.zeros_like(sum_ref)

    sum_ref[...] += jnp.broadcast_to(bsum, sum_ref.shape)

    @pl.when(i == NBLK - 1)
    def _():
        g = sum_ref[0:1, :] * (1.0 / N)
        gmu = jnp.dot(
            jnp.maximum(
                jnp.dot(g, gm1_ref[...],
                        preferred_element_type=jnp.float32) + gb1_ref[...],
                0.0),
            gm2_ref[...], preferred_element_type=jnp.float32) + gb2_ref[...]
        glv = jnp.dot(
            jnp.maximum(
                jnp.dot(g, gl1_ref[...],
                        preferred_element_type=jnp.float32) + gc1_ref[...],
                0.0),
            gl2_ref[...], preferred_element_type=jnp.float32) + gc2_ref[...]
        mug_ref[...] = jnp.broadcast_to(gmu, mug_ref.shape)
        lvg_ref[...] = jnp.broadcast_to(glv, lvg_ref.shape)


def _row_spec(cols):
    return pl.BlockSpec((BN, cols), lambda i: (i, 0))


def _w_spec(shape):
    nd = len(shape)
    return pl.BlockSpec(shape, lambda i, _nd=nd: (0,) * _nd)


def kernel(x, edge_index, input_W, input_b, msg_W, msg_b, upd_W, upd_b,
           gru_Wih, gru_Whh, gru_bih, gru_bhh, mu_W, mu_b, lv_W, lv_b,
           gmu_W1, gmu_b1, gmu_W2, gmu_b2, glv_W1, glv_b1, glv_W2, glv_b2):
    f32 = jnp.float32
    src = edge_index[0]
    dst = edge_index[1]

    wih_t = gru_Wih.T
    whh_t = gru_Whh.T
    wr, wz, wn = wih_t[:, :S], wih_t[:, S:2 * S], wih_t[:, 2 * S:]
    hr, hz, hn = whh_t[:, :S], whh_t[:, S:2 * S], whh_t[:, 2 * S:]
    br = gru_bih[:S].reshape(1, S)
    bz = gru_bih[S:2 * S].reshape(1, S)
    bn = gru_bih[2 * S:].reshape(1, S)
    bhr = gru_bhh[:S].reshape(1, S)
    bhz = gru_bhh[S:2 * S].reshape(1, S)
    bhn = gru_bhh[2 * S:].reshape(1, S)

    src_pad = jnp.concatenate(
        [src, jnp.zeros((E_PAD - E,), jnp.int32)])
    dst_pad = jnp.concatenate(
        [dst, jnp.full((E_PAD - E,), -1, jnp.int32)])

    dst2 = dst_pad.reshape(640, 1280)
    dl0, dl1 = pl.pallas_call(
        _tc_dl_body,
        grid=(1,),
        in_specs=[pl.BlockSpec((640, 1280), lambda i: (0, 0))],
        out_specs=[pl.BlockSpec((640, 1280), lambda i: (0, 0))] * 2,
        out_shape=[jax.ShapeDtypeStruct((640, 1280), jnp.int32)] * 2,
    )(dst2)
    dl_flat = jnp.concatenate([dl0.reshape(-1), dl1.reshape(-1)])

    csrc, cdst_flat, cnts = _sc_compact(src_pad, dl_flat)
    cdst2 = cdst_flat.reshape(NC * NS * (EPT // CH), CH)
    zrows = jnp.zeros((TROWS, S), jnp.float32)

    h, m = pl.pallas_call(
        _tc_init_body,
        grid=(NBLK,),
        in_specs=[
            _row_spec(IN),
            _w_spec((IN, S)), _w_spec((1, S)),
            _w_spec((S, S)), _w_spec((1, S)),
        ],
        out_specs=[_row_spec(S), _row_spec(S)],
        out_shape=[
            jax.ShapeDtypeStruct((N, S), f32),
            jax.ShapeDtypeStruct((N, S), f32),
        ],
    )(x, input_W, input_b.reshape(1, S), msg_W[0], msg_b[0].reshape(1, S))

    round_specs = (
        [_row_spec(S), _row_spec(S)]
        + [_w_spec((S, S)), _w_spec((1, S))] * 4
        + [_w_spec((S, S))] * 3
        + [_w_spec((1, S))] * 3
    )

    for r in range(R):
        agg_pad = _sc_agg(m, csrc, cdst2, cnts, zrows)
        agg = jnp.concatenate(
            [agg_pad[:HALF], agg_pad[HALF_PAD:HALF_PAD + HALF]], axis=0)
        round_args = (
            agg, h,
            upd_W[r], upd_b[r].reshape(1, S),
            wr, br, wz, bz, wn, bn,
            hr, hz, hn, bhr, bhz, bhn,
        )
        if r < R - 1:
            h, m = pl.pallas_call(
                _tc_round_body,
                grid=(NBLK,),
                in_specs=round_specs + [_w_spec((S, S)), _w_spec((1, S))],
                out_specs=[_row_spec(S), _row_spec(S)],
                out_shape=[
                    jax.ShapeDtypeStruct((N, S), f32),
                    jax.ShapeDtypeStruct((N, S), f32),
                ],
            )(*round_args, msg_W[r + 1], msg_b[r + 1].reshape(1, S))
        else:
            mu_node, lv_node, mu_g8, lv_g8 = pl.pallas_call(
                _tc_last_body,
                grid=(NBLK,),
                in_specs=round_specs + [
                    _w_spec((S, Z)), _w_spec((1, Z)),
                    _w_spec((S, Z)), _w_spec((1, Z)),
                    _w_spec((S, S)), _w_spec((1, S)),
                    _w_spec((S, G)), _w_spec((1, G)),
                    _w_spec((S, S)), _w_spec((1, S)),
                    _w_spec((S, G)), _w_spec((1, G)),
                ],
                out_specs=[
                    _row_spec(Z), _row_spec(Z),
                    pl.BlockSpec((8, G), lambda i: (0, 0)),
                    pl.BlockSpec((8, G), lambda i: (0, 0)),
                ],
                out_shape=[
                    jax.ShapeDtypeStruct((N, Z), f32),
                    jax.ShapeDtypeStruct((N, Z), f32),
                    jax.ShapeDtypeStruct((8, G), f32),
                    jax.ShapeDtypeStruct((8, G), f32),
                ],
                scratch_shapes=[pltpu.VMEM((8, S), f32)],
            )(*round_args,
              mu_W, mu_b.reshape(1, Z), lv_W, lv_b.reshape(1, Z),
              gmu_W1, gmu_b1.reshape(1, S), gmu_W2, gmu_b2.reshape(1, G),
              glv_W1, glv_b1.reshape(1, S), glv_W2, glv_b2.reshape(1, G))

    return (mu_node, lv_node, mu_g8[0], lv_g8[0])


# speedup vs baseline: 1.4469x; 1.4411x over previous
import functools

import jax
import jax.numpy as jnp
from jax import lax
from jax.experimental import pallas as pl
from jax.experimental.pallas import tpu as pltpu
from jax.experimental.pallas import tpu_sc as plsc

N = 50000
E = 800000
IN = 128
S = 64
Z = 32
G = 16
R = 6

NC = 2
NS = 16
HALF = 25000
HALF_PAD = 25008
TROWS = HALF_PAD // NS
TRASH = HALF_PAD - 1
CH = 64
E_PAD = 819200
EPT = E_PAD // NS
CPB = 16
RING = 7
LOOK = 6
IB = CH * CPB
NBI = EPT // IB
DROWS = E_PAD // CH
ZROWS = 16

_sc_mesh = plsc.VectorSubcoreMesh(core_axis_name="c", subcore_axis_name="s")

_TOTC = NC * NS * EPT


@functools.partial(
    pl.kernel,
    out_type=[
        jax.ShapeDtypeStruct((_TOTC,), jnp.int32),
        jax.ShapeDtypeStruct((_TOTC,), jnp.int32),
        jax.ShapeDtypeStruct((NC * NS * 16,), jnp.int32),
    ],
    mesh=_sc_mesh,
    scratch_types=[
        pltpu.VMEM((IB,), jnp.int32),
        pltpu.VMEM((IB,), jnp.int32),
        pltpu.VMEM((EPT + 16,), jnp.int32),
        pltpu.VMEM((EPT + 16,), jnp.int32),
    ],
    compiler_params=pltpu.CompilerParams(
        use_tc_tiling_on_sc=False, needs_layout_passes=False),
)
def _sc_compact(src_hbm, dl_hbm, csrc_hbm, cdst_hbm, cnt_hbm,
                srcst, dlst, csb, cdb):
    c = lax.axis_index("c")
    s = lax.axis_index("s")
    w = c * NS + s

    def _blk(b, off):
        pltpu.sync_copy(src_hbm.at[pl.ds(s * EPT + b * IB, IB)], srcst)
        pltpu.sync_copy(dl_hbm.at[pl.ds(c * E_PAD + s * EPT + b * IB, IB)],
                        dlst)

        def _grp(g, off2):
            s16 = srcst[pl.ds(g * 16, 16)]
            d16 = dlst[pl.ds(g * 16, 16)]
            keep = d16 != TRASH
            ki = jnp.where(keep, 1, 0)
            pos = jnp.where(keep, off2 + jnp.cumsum(ki) - 1, EPT)
            plsc.store_scatter(csb, [pos], s16)
            plsc.store_scatter(cdb, [pos], d16)
            return off2 + jnp.sum(ki)

        return lax.fori_loop(0, IB // 16, _grp, off)

    cnt = lax.fori_loop(0, NBI, _blk, jnp.int32(0))

    target = jnp.maximum((cnt + IB - 1) // IB, 1) * IB
    zpad = jnp.zeros((16,), jnp.int32)
    tpad = jnp.full((16,), TRASH, jnp.int32)

    def _pad(p, off2):
        csb[pl.ds(off2, 16)] = zpad
        cdb[pl.ds(off2, 16)] = tpad
        return off2 + 16

    lax.fori_loop(0, (target - cnt + 15) // 16, _pad, cnt)

    nblk = target // IB
    srcst[pl.ds(0, 16)] = jnp.full((16,), nblk, jnp.int32)
    pltpu.sync_copy(srcst.at[pl.ds(0, 16)], cnt_hbm.at[pl.ds(w * 16, 16)])
    pltpu.sync_copy(csb.at[pl.ds(0, EPT)], csrc_hbm.at[pl.ds(w * EPT, EPT)])
    pltpu.sync_copy(cdb.at[pl.ds(0, EPT)], cdst_hbm.at[pl.ds(w * EPT, EPT)])


@functools.partial(
    pl.kernel,
    out_type=jax.ShapeDtypeStruct((NC * HALF_PAD, S), jnp.float32),
    mesh=_sc_mesh,
    scratch_types=[
        pltpu.VMEM((IB,), jnp.int32),
        pltpu.VMEM((CPB, CH), jnp.int32),
        pltpu.VMEM((RING, CH, S), jnp.float32),
        pltpu.VMEM((16,), jnp.int32),
        pltpu.VMEM_SHARED((HALF_PAD, S), jnp.float32),
        pltpu.SemaphoreType.DMA((RING,)),
        pltpu.SemaphoreType.DMA((RING,)),
        pltpu.SemaphoreType.DMA,
    ],
    compiler_params=pltpu.CompilerParams(
        use_tc_tiling_on_sc=False, needs_layout_passes=False),
)
def _sc_agg(m_hbm, csrc_hbm, cdst_hbm, cnt_hbm, z_hbm, out_hbm, srcb, dstb,
            rows, cntv, aggsh, semg, semsc, semi):
    c = lax.axis_index("c")
    s = lax.axis_index("s")
    w = c * NS + s

    pltpu.sync_copy(z_hbm, aggsh.at[pl.ds(s * TROWS, TROWS)])
    pltpu.sync_copy(cnt_hbm.at[pl.ds(w * 16, 16)], cntv)
    plsc.subcore_barrier()
    nblk = jnp.max(cntv[...])

    ebase = w * EPT
    dlbase = w * (EPT // CH)

    def _fire_src(blk):
        pltpu.async_copy(csrc_hbm.at[pl.ds(ebase + blk * IB, IB)], srcb,
                         semi)

    def _fire_dst(blk):
        pltpu.async_copy(cdst_hbm.at[pl.ds(dlbase + blk * CPB, CPB)], dstb,
                         semi)

    def _wait_idx():
        pltpu.make_async_copy(csrc_hbm.at[pl.ds(0, IB)], srcb, semi).wait()
        pltpu.make_async_copy(cdst_hbm.at[pl.ds(0, CPB)], dstb, semi).wait()

    def _gather(k):
        q = k % RING
        return pltpu.async_copy(
            m_hbm.at[srcb.at[pl.ds(k * CH, CH)]], rows.at[q], semg.at[q])

    _fire_src(0)
    _fire_dst(0)

    def _blk_body(blk, carry):
        _wait_idx()
        descs = [None] * CPB
        g = [None] * CPB
        waited = set()
        for k in range(LOOK):
            g[k] = _gather(k)
        for k in range(CPB):
            q = k % RING
            g[k].wait()
            descs[k] = pltpu.async_copy(rows.at[q], aggsh.at[dstb.at[k]],
                                        semsc.at[q], add=True)
            if k + LOOK < CPB:
                if k + LOOK - RING >= 0:
                    descs[k + LOOK - RING].wait()
                    waited.add(k + LOOK - RING)
                g[k + LOOK] = _gather(k + LOOK)

        @pl.when(blk + 1 < nblk)
        def _():
            _fire_src(blk + 1)

        for k in range(CPB):
            if k not in waited:
                descs[k].wait()

        @pl.when(blk + 1 < nblk)
        def _():
            _fire_dst(blk + 1)

        return carry

    lax.fori_loop(0, nblk, _blk_body, 0)

    plsc.subcore_barrier()

    pltpu.sync_copy(
        aggsh.at[pl.ds(s * TROWS, TROWS)],
        out_hbm.at[pl.ds(c * HALF_PAD + s * TROWS, TROWS)],
    )


BN = 2000
NBLK = N // BN


def _tc_dl_body(d_ref, dl0_ref, dl1_ref):
    d = d_ref[...]
    dl0_ref[...] = jnp.where((d >= 0) & (d < HALF), d, TRASH)
    dl1_ref[...] = jnp.where(d >= HALF, d - HALF, TRASH)


def _tc_init_body(x_ref, wi_ref, bi_ref, wm_ref, bm_ref, h_ref, m_ref):
    h = jnp.maximum(
        jnp.dot(x_ref[...], wi_ref[...],
                preferred_element_type=jnp.float32) + bi_ref[...], 0.0)
    h_ref[...] = h
    m_ref[...] = jnp.maximum(
        jnp.dot(h, wm_ref[...], preferred_element_type=jnp.float32)
        + bm_ref[...], 0.0)


def _gru_from_agg(agg, h, wu, bu, wr, br, wz, bz, wn, bn, hr, hz, hn_w,
                  bhr, bhz, bhn):
    msg = jnp.maximum(
        jnp.dot(agg, wu, preferred_element_type=jnp.float32) + bu, 0.0)
    g_r = jax.nn.sigmoid(
        jnp.dot(msg, wr, preferred_element_type=jnp.float32) + br
        + jnp.dot(h, hr, preferred_element_type=jnp.float32) + bhr)
    g_z = jax.nn.sigmoid(
        jnp.dot(msg, wz, preferred_element_type=jnp.float32) + bz
        + jnp.dot(h, hz, preferred_element_type=jnp.float32) + bhz)
    g_n = jnp.tanh(
        jnp.dot(msg, wn, preferred_element_type=jnp.float32) + bn
        + g_r * (jnp.dot(h, hn_w, preferred_element_type=jnp.float32) + bhn))
    return (1.0 - g_z) * g_n + g_z * h


def _tc_round_body(agg_ref, h_ref, wu_ref, bu_ref, wr_ref, br_ref, wz_ref,
                   bz_ref, wn_ref, bn_ref, hr_ref, hz_ref, hn_ref, bhr_ref,
                   bhz_ref, bhn_ref, wm_ref, bm_ref, hout_ref, mout_ref):
    hn = _gru_from_agg(agg_ref[...], h_ref[...], wu_ref[...], bu_ref[...],
                       wr_ref[...], br_ref[...], wz_ref[...], bz_ref[...],
                       wn_ref[...], bn_ref[...], hr_ref[...], hz_ref[...],
                       hn_ref[...], bhr_ref[...], bhz_ref[...], bhn_ref[...])
    hout_ref[...] = hn
    mout_ref[...] = jnp.maximum(
        jnp.dot(hn, wm_ref[...], preferred_element_type=jnp.float32)
        + bm_ref[...], 0.0)


def _tc_last_body(agg_ref, h_ref, wu_ref, bu_ref, wr_ref, br_ref, wz_ref,
                  bz_ref, wn_ref, bn_ref, hr_ref, hz_ref, hn_ref, bhr_ref,
                  bhz_ref, bhn_ref, muw_ref, mub_ref, lvw_ref, lvb_ref,
                  gm1_ref, gb1_ref, gm2_ref, gb2_ref, gl1_ref, gc1_ref,
                  gl2_ref, gc2_ref, mu_ref, lv_ref, mug_ref, lvg_ref,
                  sum_ref):
    i = pl.program_id(0)
    hn = _gru_from_agg(agg_ref[...], h_ref[...], wu_ref[...], bu_ref[...],
                       wr_ref[...], br_ref[...], wz_ref[...], bz_ref[...],
                       wn_ref[...], bn_ref[...], hr_ref[...], hz_ref[...],
                       hn_ref[...], bhr_ref[...], bhz_ref[...], bhn_ref[...])
    mu_ref[...] = jnp.dot(
        hn, muw_ref[...], preferred_element_type=jnp.float32) + mub_ref[...]
    lv_ref[...] = jnp.dot(
        hn, lvw_ref[...], preferred_element_type=jnp.float32) + lvb_ref[...]

    bsum = jnp.sum(hn, axis=0, keepdims=True)

    @pl.when(i == 0)
    def _():
        sum_ref[...] = jnp.zeros_like(sum_ref)

    sum_ref[...] += jnp.broadcast_to(bsum, sum_ref.shape)

    @pl.when(i == NBLK - 1)
    def _():
        g = sum_ref[0:1, :] * (1.0 / N)
        gmu = jnp.dot(
            jnp.maximum(
                jnp.dot(g, gm1_ref[...],
                        preferred_element_type=jnp.float32) + gb1_ref[...],
                0.0),
            gm2_ref[...], preferred_element_type=jnp.float32) + gb2_ref[...]
        glv = jnp.dot(
            jnp.maximum(
                jnp.dot(g, gl1_ref[...],
                        preferred_element_type=jnp.float32) + gc1_ref[...],
                0.0),
            gl2_ref[...], preferred_element_type=jnp.float32) + gc2_ref[...]
        mug_ref[...] = jnp.broadcast_to(gmu, mug_ref.shape)
        lvg_ref[...] = jnp.broadcast_to(glv, lvg_ref.shape)


def _row_spec(cols):
    return pl.BlockSpec((BN, cols), lambda i: (i, 0))


def _w_spec(shape):
    nd = len(shape)
    return pl.BlockSpec(shape, lambda i, _nd=nd: (0,) * _nd)


def kernel(x, edge_index, input_W, input_b, msg_W, msg_b, upd_W, upd_b,
           gru_Wih, gru_Whh, gru_bih, gru_bhh, mu_W, mu_b, lv_W, lv_b,
           gmu_W1, gmu_b1, gmu_W2, gmu_b2, glv_W1, glv_b1, glv_W2, glv_b2):
    f32 = jnp.float32
    src = edge_index[0]
    dst = edge_index[1]

    wih_t = gru_Wih.T
    whh_t = gru_Whh.T
    wr, wz, wn = wih_t[:, :S], wih_t[:, S:2 * S], wih_t[:, 2 * S:]
    hr, hz, hn = whh_t[:, :S], whh_t[:, S:2 * S], whh_t[:, 2 * S:]
    br = gru_bih[:S].reshape(1, S)
    bz = gru_bih[S:2 * S].reshape(1, S)
    bn = gru_bih[2 * S:].reshape(1, S)
    bhr = gru_bhh[:S].reshape(1, S)
    bhz = gru_bhh[S:2 * S].reshape(1, S)
    bhn = gru_bhh[2 * S:].reshape(1, S)

    src_pad = jnp.concatenate(
        [src, jnp.zeros((E_PAD - E,), jnp.int32)])
    dst_pad = jnp.concatenate(
        [dst, jnp.full((E_PAD - E,), -1, jnp.int32)])

    dst2 = dst_pad.reshape(640, 1280)
    dl0, dl1 = pl.pallas_call(
        _tc_dl_body,
        grid=(1,),
        in_specs=[pl.BlockSpec((640, 1280), lambda i: (0, 0))],
        out_specs=[pl.BlockSpec((640, 1280), lambda i: (0, 0))] * 2,
        out_shape=[jax.ShapeDtypeStruct((640, 1280), jnp.int32)] * 2,
    )(dst2)
    dl_flat = jnp.concatenate([dl0.reshape(-1), dl1.reshape(-1)])

    csrc, cdst_flat, cnts = _sc_compact(src_pad, dl_flat)
    cdst2 = cdst_flat.reshape(NC * NS * (EPT // CH), CH)
    zrows = jnp.zeros((TROWS, S), jnp.float32)

    h, m = pl.pallas_call(
        _tc_init_body,
        grid=(NBLK,),
        in_specs=[
            _row_spec(IN),
            _w_spec((IN, S)), _w_spec((1, S)),
            _w_spec((S, S)), _w_spec((1, S)),
        ],
        out_specs=[_row_spec(S), _row_spec(S)],
        out_shape=[
            jax.ShapeDtypeStruct((N, S), f32),
            jax.ShapeDtypeStruct((N, S), f32),
        ],
    )(x, input_W, input_b.reshape(1, S), msg_W[0], msg_b[0].reshape(1, S))

    round_specs = (
        [_row_spec(S), _row_spec(S)]
        + [_w_spec((S, S)), _w_spec((1, S))] * 4
        + [_w_spec((S, S))] * 3
        + [_w_spec((1, S))] * 3
    )

    for r in range(R):
        agg_pad = _sc_agg(m, csrc, cdst2, cnts, zrows)
        agg = jnp.concatenate(
            [agg_pad[:HALF], agg_pad[HALF_PAD:HALF_PAD + HALF]], axis=0)
        round_args = (
            agg, h,
            upd_W[r], upd_b[r].reshape(1, S),
            wr, br, wz, bz, wn, bn,
            hr, hz, hn, bhr, bhz, bhn,
        )
        if r < R - 1:
            h, m = pl.pallas_call(
                _tc_round_body,
                grid=(NBLK,),
                in_specs=round_specs + [_w_spec((S, S)), _w_spec((1, S))],
                out_specs=[_row_spec(S), _row_spec(S)],
                out_shape=[
                    jax.ShapeDtypeStruct((N, S), f32),
                    jax.ShapeDtypeStruct((N, S), f32),
                ],
            )(*round_args, msg_W[r + 1], msg_b[r + 1].reshape(1, S))
        else:
            mu_node, lv_node, mu_g8, lv_g8 = pl.pallas_call(
                _tc_last_body,
                grid=(NBLK,),
                in_specs=round_specs + [
                    _w_spec((S, Z)), _w_spec((1, Z)),
                    _w_spec((S, Z)), _w_spec((1, Z)),
                    _w_spec((S, S)), _w_spec((1, S)),
                    _w_spec((S, G)), _w_spec((1, G)),
                    _w_spec((S, S)), _w_spec((1, S)),
                    _w_spec((S, G)), _w_spec((1, G)),
                ],
                out_specs=[
                    _row_spec(Z), _row_spec(Z),
                    pl.BlockSpec((8, G), lambda i: (0, 0)),
                    pl.BlockSpec((8, G), lambda i: (0, 0)),
                ],
                out_shape=[
                    jax.ShapeDtypeStruct((N, Z), f32),
                    jax.ShapeDtypeStruct((N, Z), f32),
                    jax.ShapeDtypeStruct((8, G), f32),
                    jax.ShapeDtypeStruct((8, G), f32),
                ],
                scratch_shapes=[pltpu.VMEM((8, S), f32)],
            )(*round_args,
              mu_W, mu_b.reshape(1, Z), lv_W, lv_b.reshape(1, Z),
              gmu_W1, gmu_b1.reshape(1, S), gmu_W2, gmu_b2.reshape(1, G),
              glv_W1, glv_b1.reshape(1, S), glv_W2, glv_b2.reshape(1, G))

    return (mu_node, lv_node, mu_g8[0], lv_g8[0])
